# baseline (speedup 1.0000x reference)
import functools

import jax
import jax.numpy as jnp
from jax.experimental import pallas as pl


def _known_body(x_ref, cat_ref, tab_ref, wt_ref, b_ref, out_ref, *, vocab):
    BB, TB, n_real = x_ref.shape
    n_cat = cat_ref.shape[-1]
    L = tab_ref.shape[-1]
    rows = BB * TB
    x = x_ref[...]
    dense = x[:, :, None, :] * wt_ref[...][None, None, :, :]
    cat2 = cat_ref[...].reshape(rows, n_cat)
    iota_v = jax.lax.broadcasted_iota(jnp.int32, (rows, vocab), 1)
    g4 = []
    for j in range(n_cat):
        onehot = (cat2[:, j:j + 1] == iota_v).astype(jnp.float32)
        g = jnp.dot(onehot, tab_ref[j], preferred_element_type=jnp.float32)
        g4.append(g.reshape(BB, TB, L, 1))
    tile = jnp.concatenate([dense] + g4, axis=-1)
    out_ref[...] = tile + b_ref[...][None, None, :, :]


def _obs_body(x_ref, wt_ref, b_ref, out_ref):
    out_ref[...] = (x_ref[...][:, :, None, :] * wt_ref[...][None, None, :, :]
                    + b_ref[...][None, None, :, :])


def _static_body(idx_ref, tab_ref, out_ref, *, vocab):
    rblk = idx_ref.shape[0]
    iota_v = jax.lax.broadcasted_iota(jnp.int32, (rblk, vocab), 1)
    for i in range(tab_ref.shape[0]):
        onehot = (idx_ref[:, i:i + 1] == iota_v).astype(jnp.float32)
        out_ref[:, i, :] = jnp.dot(onehot, tab_ref[i],
                                   preferred_element_type=jnp.float32)


def kernel(static, known_real, known_categorical, observed, static_tables,
           known_cat_tables, real_W, real_b, obs_W, obs_b):
    B, T, n_real = known_real.shape
    n_obs = observed.shape[-1]
    n_cat = known_categorical.shape[-1]
    n_static = static_tables.shape[0]
    vocab, L = static_tables.shape[1], static_tables.shape[2]
    n_known = n_real + n_cat

    cat = known_categorical.astype(jnp.int32)
    wk_t = real_W.T
    bk_t = jnp.concatenate([real_b.T, jnp.zeros((L, n_cat), real_b.dtype)],
                           axis=1)
    wo_t = obs_W.T
    bo_t = obs_b.T

    full = lambda shape: pl.BlockSpec(shape, lambda *a: (0,) * len(shape))
    BB = 2
    known_embs = pl.pallas_call(
        functools.partial(_known_body, vocab=vocab),
        grid=(B // BB,),
        in_specs=[
            pl.BlockSpec((BB, T, n_real), lambda r: (r, 0, 0)),
            pl.BlockSpec((BB, T, n_cat), lambda r: (r, 0, 0)),
            full((n_cat, vocab, L)),
            full((L, n_real)),
            full((L, n_known)),
        ],
        out_specs=pl.BlockSpec((BB, T, L, n_known), lambda r: (r, 0, 0, 0)),
        out_shape=jax.ShapeDtypeStruct((B, T, L, n_known), jnp.float32),
    )(known_real, cat, known_cat_tables, wk_t, bk_t)

    OB = 4
    obs_embs = pl.pallas_call(
        _obs_body,
        grid=(B // OB,),
        in_specs=[
            pl.BlockSpec((OB, T, n_obs), lambda r: (r, 0, 0)),
            full((L, n_obs)),
            full((L, n_obs)),
        ],
        out_specs=pl.BlockSpec((OB, T, L, n_obs), lambda r: (r, 0, 0, 0)),
        out_shape=jax.ShapeDtypeStruct((B, T, L, n_obs), jnp.float32),
    )(observed, wo_t, bo_t)

    sidx = static[:, 0, :].astype(jnp.int32)
    SBLK = 256 if B % 256 == 0 else B
    static_embs = pl.pallas_call(
        functools.partial(_static_body, vocab=vocab),
        grid=(B // SBLK,),
        in_specs=[
            pl.BlockSpec((SBLK, n_static), lambda r: (r, 0)),
            full((n_static, vocab, L)),
        ],
        out_specs=pl.BlockSpec((SBLK, n_static, L), lambda r: (r, 0, 0)),
        out_shape=jax.ShapeDtypeStruct((B, n_static, L), jnp.float32),
    )(sidx, static_tables)

    return (static_embs, known_embs, obs_embs)


# speedup vs baseline: 10.7626x; 10.7626x over previous
import functools

import jax
import jax.numpy as jnp
from jax.experimental import pallas as pl


def _known_body(x_ref, cat_ref, tab_ref, w_ref, b_ref, out_ref, *, vocab):
    T, n_real, BB = x_ref.shape[0], x_ref.shape[1], x_ref.shape[2]
    n_cat = cat_ref.shape[-1]
    L = tab_ref.shape[-1]
    out_ref[:, :n_real, :, :] = (x_ref[...] * w_ref[...]
                                 + b_ref[...])
    iota_v = jax.lax.broadcasted_iota(jnp.int32, (T, BB, vocab), 2)
    for j in range(n_cat):
        idx = cat_ref[:, :, j:j + 1]
        onehot = (idx == iota_v).astype(jnp.float32).reshape(T * BB, vocab)
        g = jnp.dot(onehot, tab_ref[j], preferred_element_type=jnp.float32)
        out_ref[:, n_real + j, :, :] = g.reshape(T, BB, L)


def _obs_body(x_ref, w_ref, b_ref, out_ref):
    out_ref[...] = (x_ref[...] * w_ref[...][None, None, :, :]
                    + b_ref[...][None, None, :, :])


def _static_body(idx_ref, tab_ref, out_ref, *, vocab):
    rblk = idx_ref.shape[0]
    iota_v = jax.lax.broadcasted_iota(jnp.int32, (rblk, vocab), 1)
    for i in range(tab_ref.shape[0]):
        onehot = (idx_ref[:, i:i + 1] == iota_v).astype(jnp.float32)
        out_ref[:, i, :] = jnp.dot(onehot, tab_ref[i],
                                   preferred_element_type=jnp.float32)


def kernel(static, known_real, known_categorical, observed, static_tables,
           known_cat_tables, real_W, real_b, obs_W, obs_b):
    B, T, n_real = known_real.shape
    n_obs = observed.shape[-1]
    n_cat = known_categorical.shape[-1]
    n_static = static_tables.shape[0]
    vocab, L = static_tables.shape[1], static_tables.shape[2]
    n_known = n_real + n_cat

    xk_t = known_real.transpose(1, 2, 0)[..., None]
    cat_t = known_categorical.transpose(1, 0, 2).astype(jnp.int32)
    xo4 = observed[..., None]
    w4 = real_W[None, :, None, :]
    b4 = real_b[None, :, None, :]

    full = lambda shape: pl.BlockSpec(shape, lambda *a: (0,) * len(shape))
    BB = 8
    known_p = pl.pallas_call(
        functools.partial(_known_body, vocab=vocab),
        grid=(B // BB,),
        in_specs=[
            pl.BlockSpec((T, n_real, BB, 1), lambda r: (0, 0, r, 0)),
            pl.BlockSpec((T, BB, n_cat), lambda r: (0, r, 0)),
            full((n_cat, vocab, L)),
            full((1, n_real, 1, L)),
            full((1, n_real, 1, L)),
        ],
        out_specs=pl.BlockSpec((T, n_known, BB, L), lambda r: (0, 0, r, 0)),
        out_shape=jax.ShapeDtypeStruct((T, n_known, B, L), jnp.float32),
    )(xk_t, cat_t, known_cat_tables, w4, b4)

    OB = 8
    obs_p = pl.pallas_call(
        _obs_body,
        grid=(B // OB,),
        in_specs=[
            pl.BlockSpec((OB, T, n_obs, 1), lambda r: (r, 0, 0, 0)),
            full((n_obs, L)),
            full((n_obs, L)),
        ],
        out_specs=pl.BlockSpec((OB, T, n_obs, L), lambda r: (r, 0, 0, 0)),
        out_shape=jax.ShapeDtypeStruct((B, T, n_obs, L), jnp.float32),
    )(xo4, obs_W, obs_b)

    sidx = static[:, 0, :].astype(jnp.int32)
    SBLK = 256 if B % 256 == 0 else B
    static_embs = pl.pallas_call(
        functools.partial(_static_body, vocab=vocab),
        grid=(B // SBLK,),
        in_specs=[
            pl.BlockSpec((SBLK, n_static), lambda r: (r, 0)),
            full((n_static, vocab, L)),
        ],
        out_specs=pl.BlockSpec((SBLK, n_static, L), lambda r: (r, 0, 0)),
        out_shape=jax.ShapeDtypeStruct((B, n_static, L), jnp.float32),
    )(sidx, static_tables)

    return (static_embs,
            jnp.transpose(known_p, (2, 0, 3, 1)),
            jnp.swapaxes(obs_p, 2, 3))


# speedup vs baseline: 12.3942x; 1.1516x over previous
import functools

import jax
import jax.numpy as jnp
from jax.experimental import pallas as pl


def _known_body(x_ref, cat_ref, tab_ref, w_ref, b_ref, out_ref, *, vocab):
    T, n_real, BB = x_ref.shape[0], x_ref.shape[1], x_ref.shape[2]
    n_cat = cat_ref.shape[-1]
    L = tab_ref.shape[-1]
    out_ref[:, :n_real, :, :] = (x_ref[...] * w_ref[...]
                                 + b_ref[...])
    iota_v = jax.lax.broadcasted_iota(jnp.int32, (T, BB, vocab), 2)
    for j in range(n_cat):
        idx = cat_ref[:, :, j:j + 1]
        onehot = (idx == iota_v).astype(jnp.float32).reshape(T * BB, vocab)
        g = jnp.dot(onehot, tab_ref[j], preferred_element_type=jnp.float32)
        out_ref[:, n_real + j, :, :] = g.reshape(T, BB, L)


def _obs_body(x_ref, w_ref, b_ref, out_ref):
    out_ref[...] = (x_ref[...] * w_ref[...][None, None, :, :]
                    + b_ref[...][None, None, :, :])


def _static_body(idx_ref, tab_ref, out_ref, *, vocab):
    rblk = idx_ref.shape[0]
    iota_v = jax.lax.broadcasted_iota(jnp.int32, (rblk, vocab), 1)
    for i in range(tab_ref.shape[0]):
        onehot = (idx_ref[:, i:i + 1] == iota_v).astype(jnp.float32)
        out_ref[:, i, :] = jnp.dot(onehot, tab_ref[i],
                                   preferred_element_type=jnp.float32)


def kernel(static, known_real, known_categorical, observed, static_tables,
           known_cat_tables, real_W, real_b, obs_W, obs_b):
    B, T, n_real = known_real.shape
    n_obs = observed.shape[-1]
    n_cat = known_categorical.shape[-1]
    n_static = static_tables.shape[0]
    vocab, L = static_tables.shape[1], static_tables.shape[2]
    n_known = n_real + n_cat

    xk_t = known_real.transpose(1, 2, 0)[..., None]
    cat_t = known_categorical.transpose(1, 0, 2).astype(jnp.int32)
    xo4 = observed[..., None]
    w4 = real_W[None, :, None, :]
    b4 = real_b[None, :, None, :]

    full = lambda shape: pl.BlockSpec(shape, lambda *a: (0,) * len(shape))
    BB = 16
    known_p = pl.pallas_call(
        functools.partial(_known_body, vocab=vocab),
        grid=(B // BB,),
        in_specs=[
            pl.BlockSpec((T, n_real, BB, 1), lambda r: (0, 0, r, 0)),
            pl.BlockSpec((T, BB, n_cat), lambda r: (0, r, 0)),
            full((n_cat, vocab, L)),
            full((1, n_real, 1, L)),
            full((1, n_real, 1, L)),
        ],
        out_specs=pl.BlockSpec((T, n_known, BB, L), lambda r: (0, 0, r, 0)),
        out_shape=jax.ShapeDtypeStruct((T, n_known, B, L), jnp.float32),
    )(xk_t, cat_t, known_cat_tables, w4, b4)

    OB = 16
    obs_p = pl.pallas_call(
        _obs_body,
        grid=(B // OB,),
        in_specs=[
            pl.BlockSpec((OB, T, n_obs, 1), lambda r: (r, 0, 0, 0)),
            full((n_obs, L)),
            full((n_obs, L)),
        ],
        out_specs=pl.BlockSpec((OB, T, n_obs, L), lambda r: (r, 0, 0, 0)),
        out_shape=jax.ShapeDtypeStruct((B, T, n_obs, L), jnp.float32),
    )(xo4, obs_W, obs_b)

    sidx = static[:, 0, :].astype(jnp.int32)
    SBLK = 256 if B % 256 == 0 else B
    static_embs = pl.pallas_call(
        functools.partial(_static_body, vocab=vocab),
        grid=(B // SBLK,),
        in_specs=[
            pl.BlockSpec((SBLK, n_static), lambda r: (r, 0)),
            full((n_static, vocab, L)),
        ],
        out_specs=pl.BlockSpec((SBLK, n_static, L), lambda r: (r, 0, 0)),
        out_shape=jax.ShapeDtypeStruct((B, n_static, L), jnp.float32),
    )(sidx, static_tables)

    return (static_embs,
            jnp.transpose(known_p, (2, 0, 3, 1)),
            jnp.swapaxes(obs_p, 2, 3))


# speedup vs baseline: 12.7652x; 1.0299x over previous
import functools

import jax
import jax.numpy as jnp
from jax.experimental import pallas as pl


def _known_body(x_ref, cat_ref, tab_ref, w_ref, b_ref, out_ref, *, vocab):
    T, n_real, BB = x_ref.shape[0], x_ref.shape[1], x_ref.shape[2]
    n_cat = cat_ref.shape[-1]
    L = tab_ref.shape[-1]
    out_ref[:, :n_real, :, :] = (x_ref[...] * w_ref[...]
                                 + b_ref[...])
    iota_v = jax.lax.broadcasted_iota(jnp.int32, (T, BB, vocab), 2)
    for j in range(n_cat):
        idx = cat_ref[:, :, j:j + 1]
        onehot = (idx == iota_v).astype(jnp.float32).reshape(T * BB, vocab)
        g = jnp.dot(onehot, tab_ref[j], preferred_element_type=jnp.float32)
        out_ref[:, n_real + j, :, :] = g.reshape(T, BB, L)


def _obs_body(x_ref, w_ref, b_ref, out_ref):
    out_ref[...] = (x_ref[...] * w_ref[...][None, None, :, :]
                    + b_ref[...][None, None, :, :])


def _static_body(idx_ref, tab_ref, out_ref, *, vocab):
    rblk = idx_ref.shape[0]
    iota_v = jax.lax.broadcasted_iota(jnp.int32, (rblk, vocab), 1)
    for i in range(tab_ref.shape[0]):
        onehot = (idx_ref[:, i:i + 1] == iota_v).astype(jnp.float32)
        out_ref[:, i, :] = jnp.dot(onehot, tab_ref[i],
                                   preferred_element_type=jnp.float32)


def kernel(static, known_real, known_categorical, observed, static_tables,
           known_cat_tables, real_W, real_b, obs_W, obs_b):
    B, T, n_real = known_real.shape
    n_obs = observed.shape[-1]
    n_cat = known_categorical.shape[-1]
    n_static = static_tables.shape[0]
    vocab, L = static_tables.shape[1], static_tables.shape[2]
    n_known = n_real + n_cat

    xk_t = known_real.transpose(1, 2, 0)[..., None]
    cat_t = known_categorical.transpose(1, 0, 2).astype(jnp.int32)
    xo4 = observed[..., None]
    w4 = real_W[None, :, None, :]
    b4 = real_b[None, :, None, :]

    full = lambda shape: pl.BlockSpec(shape, lambda *a: (0,) * len(shape))
    BB = 32
    known_p = pl.pallas_call(
        functools.partial(_known_body, vocab=vocab),
        grid=(B // BB,),
        in_specs=[
            pl.BlockSpec((T, n_real, BB, 1), lambda r: (0, 0, r, 0)),
            pl.BlockSpec((T, BB, n_cat), lambda r: (0, r, 0)),
            full((n_cat, vocab, L)),
            full((1, n_real, 1, L)),
            full((1, n_real, 1, L)),
        ],
        out_specs=pl.BlockSpec((T, n_known, BB, L), lambda r: (0, 0, r, 0)),
        out_shape=jax.ShapeDtypeStruct((T, n_known, B, L), jnp.float32),
    )(xk_t, cat_t, known_cat_tables, w4, b4)

    OB = 32
    obs_p = pl.pallas_call(
        _obs_body,
        grid=(B // OB,),
        in_specs=[
            pl.BlockSpec((OB, T, n_obs, 1), lambda r: (r, 0, 0, 0)),
            full((n_obs, L)),
            full((n_obs, L)),
        ],
        out_specs=pl.BlockSpec((OB, T, n_obs, L), lambda r: (r, 0, 0, 0)),
        out_shape=jax.ShapeDtypeStruct((B, T, n_obs, L), jnp.float32),
    )(xo4, obs_W, obs_b)

    sidx = static[:, 0, :].astype(jnp.int32)
    SBLK = 256 if B % 256 == 0 else B
    static_embs = pl.pallas_call(
        functools.partial(_static_body, vocab=vocab),
        grid=(B // SBLK,),
        in_specs=[
            pl.BlockSpec((SBLK, n_static), lambda r: (r, 0)),
            full((n_static, vocab, L)),
        ],
        out_specs=pl.BlockSpec((SBLK, n_static, L), lambda r: (r, 0, 0)),
        out_shape=jax.ShapeDtypeStruct((B, n_static, L), jnp.float32),
    )(sidx, static_tables)

    return (static_embs,
            jnp.transpose(known_p, (2, 0, 3, 1)),
            jnp.swapaxes(obs_p, 2, 3))


# speedup vs baseline: 12.8010x; 1.0028x over previous
import functools

import jax
import jax.numpy as jnp
from jax.experimental import pallas as pl


def _known_body(x_ref, cat_ref, tab_ref, w_ref, b_ref, out_ref, *, vocab):
    T, n_real, BB = x_ref.shape[0], x_ref.shape[1], x_ref.shape[2]
    n_cat = cat_ref.shape[-1]
    L = tab_ref.shape[-1]
    out_ref[:, :n_real, :, :] = (x_ref[...] * w_ref[...]
                                 + b_ref[...])
    iota_v = jax.lax.broadcasted_iota(jnp.int32, (T, BB, vocab), 2)
    for j in range(n_cat):
        idx = cat_ref[:, :, j:j + 1]
        onehot = (idx == iota_v).astype(jnp.float32).reshape(T * BB, vocab)
        g = jnp.dot(onehot, tab_ref[j], preferred_element_type=jnp.float32)
        out_ref[:, n_real + j, :, :] = g.reshape(T, BB, L)


def _obs_body(x_ref, w_ref, b_ref, out_ref):
    out_ref[...] = (x_ref[...] * w_ref[...][None, None, :, :]
                    + b_ref[...][None, None, :, :])


def _static_body(idx_ref, tab_ref, out_ref, *, vocab):
    rblk = idx_ref.shape[0]
    iota_v = jax.lax.broadcasted_iota(jnp.int32, (rblk, vocab), 1)
    for i in range(tab_ref.shape[0]):
        onehot = (idx_ref[:, i:i + 1] == iota_v).astype(jnp.float32)
        out_ref[:, i, :] = jnp.dot(onehot, tab_ref[i],
                                   preferred_element_type=jnp.float32)


def kernel(static, known_real, known_categorical, observed, static_tables,
           known_cat_tables, real_W, real_b, obs_W, obs_b):
    B, T, n_real = known_real.shape
    n_obs = observed.shape[-1]
    n_cat = known_categorical.shape[-1]
    n_static = static_tables.shape[0]
    vocab, L = static_tables.shape[1], static_tables.shape[2]
    n_known = n_real + n_cat

    xk_t = known_real.transpose(1, 2, 0)[..., None]
    cat_t = known_categorical.transpose(1, 0, 2).astype(jnp.int32)
    xo4 = observed[..., None]
    w4 = real_W[None, :, None, :]
    b4 = real_b[None, :, None, :]

    full = lambda shape: pl.BlockSpec(shape, lambda *a: (0,) * len(shape))
    BB = 32
    known_p = pl.pallas_call(
        functools.partial(_known_body, vocab=vocab),
        grid=(B // BB,),
        in_specs=[
            pl.BlockSpec((T, n_real, BB, 1), lambda r: (0, 0, r, 0)),
            pl.BlockSpec((T, BB, n_cat), lambda r: (0, r, 0)),
            full((n_cat, vocab, L)),
            full((1, n_real, 1, L)),
            full((1, n_real, 1, L)),
        ],
        out_specs=pl.BlockSpec((T, n_known, BB, L), lambda r: (0, 0, r, 0)),
        out_shape=jax.ShapeDtypeStruct((T, n_known, B, L), jnp.float32),
    )(xk_t, cat_t, known_cat_tables, w4, b4)

    OB = 64
    obs_p = pl.pallas_call(
        _obs_body,
        grid=(B // OB,),
        in_specs=[
            pl.BlockSpec((OB, T, n_obs, 1), lambda r: (r, 0, 0, 0)),
            full((n_obs, L)),
            full((n_obs, L)),
        ],
        out_specs=pl.BlockSpec((OB, T, n_obs, L), lambda r: (r, 0, 0, 0)),
        out_shape=jax.ShapeDtypeStruct((B, T, n_obs, L), jnp.float32),
    )(xo4, obs_W, obs_b)

    sidx = static[:, 0, :].astype(jnp.int32)
    SBLK = 256 if B % 256 == 0 else B
    static_embs = pl.pallas_call(
        functools.partial(_static_body, vocab=vocab),
        grid=(B // SBLK,),
        in_specs=[
            pl.BlockSpec((SBLK, n_static), lambda r: (r, 0)),
            full((n_static, vocab, L)),
        ],
        out_specs=pl.BlockSpec((SBLK, n_static, L), lambda r: (r, 0, 0)),
        out_shape=jax.ShapeDtypeStruct((B, n_static, L), jnp.float32),
    )(sidx, static_tables)

    return (static_embs,
            jnp.transpose(known_p, (2, 0, 3, 1)),
            jnp.swapaxes(obs_p, 2, 3))


# speedup vs baseline: 12.9788x; 1.0139x over previous
import functools

import jax
import jax.numpy as jnp
from jax.experimental import pallas as pl


def _known_body(x_ref, cat_ref, tab_ref, w_ref, b_ref, xo_ref, wo_ref,
                bo_ref, out_ref, obs_ref, *, vocab):
    T, n_real, BB = x_ref.shape[0], x_ref.shape[1], x_ref.shape[2]
    n_cat = cat_ref.shape[-1]
    L = tab_ref.shape[-1]
    out_ref[:, :n_real, :, :] = (x_ref[...] * w_ref[...]
                                 + b_ref[...])
    iota_v = jax.lax.broadcasted_iota(jnp.int32, (T, BB, vocab), 2)
    for j in range(n_cat):
        idx = cat_ref[:, :, j:j + 1]
        onehot = (idx == iota_v).astype(jnp.float32).reshape(T * BB, vocab)
        g = jnp.dot(onehot, tab_ref[j], preferred_element_type=jnp.float32)
        out_ref[:, n_real + j, :, :] = g.reshape(T, BB, L)
    obs_ref[...] = (xo_ref[...] * wo_ref[...][None, None, :, :]
                    + bo_ref[...][None, None, :, :])


def _obs_body(x_ref, w_ref, b_ref, out_ref):
    out_ref[...] = (x_ref[...] * w_ref[...][None, None, :, :]
                    + b_ref[...][None, None, :, :])


def _static_body(idx_ref, tab_ref, out_ref, *, vocab):
    rblk = idx_ref.shape[0]
    iota_v = jax.lax.broadcasted_iota(jnp.int32, (rblk, vocab), 1)
    for i in range(tab_ref.shape[0]):
        onehot = (idx_ref[:, i:i + 1] == iota_v).astype(jnp.float32)
        out_ref[:, i, :] = jnp.dot(onehot, tab_ref[i],
                                   preferred_element_type=jnp.float32)


def kernel(static, known_real, known_categorical, observed, static_tables,
           known_cat_tables, real_W, real_b, obs_W, obs_b):
    B, T, n_real = known_real.shape
    n_obs = observed.shape[-1]
    n_cat = known_categorical.shape[-1]
    n_static = static_tables.shape[0]
    vocab, L = static_tables.shape[1], static_tables.shape[2]
    n_known = n_real + n_cat

    xk_t = known_real.transpose(1, 2, 0)[..., None]
    cat_t = known_categorical.transpose(1, 0, 2).astype(jnp.int32)
    xo4 = observed[..., None]
    w4 = real_W[None, :, None, :]
    b4 = real_b[None, :, None, :]

    full = lambda shape: pl.BlockSpec(shape, lambda *a: (0,) * len(shape))
    BB = 16
    known_p, obs_p = pl.pallas_call(
        functools.partial(_known_body, vocab=vocab),
        grid=(B // BB,),
        in_specs=[
            pl.BlockSpec((T, n_real, BB, 1), lambda r: (0, 0, r, 0)),
            pl.BlockSpec((T, BB, n_cat), lambda r: (0, r, 0)),
            full((n_cat, vocab, L)),
            full((1, n_real, 1, L)),
            full((1, n_real, 1, L)),
            pl.BlockSpec((BB, T, n_obs, 1), lambda r: (r, 0, 0, 0)),
            full((n_obs, L)),
            full((n_obs, L)),
        ],
        out_specs=[
            pl.BlockSpec((T, n_known, BB, L), lambda r: (0, 0, r, 0)),
            pl.BlockSpec((BB, T, n_obs, L), lambda r: (r, 0, 0, 0)),
        ],
        out_shape=[
            jax.ShapeDtypeStruct((T, n_known, B, L), jnp.float32),
            jax.ShapeDtypeStruct((B, T, n_obs, L), jnp.float32),
        ],
    )(xk_t, cat_t, known_cat_tables, w4, b4, xo4, obs_W, obs_b)

    sidx = static[:, 0, :].astype(jnp.int32)
    SBLK = 256 if B % 256 == 0 else B
    static_embs = pl.pallas_call(
        functools.partial(_static_body, vocab=vocab),
        grid=(B // SBLK,),
        in_specs=[
            pl.BlockSpec((SBLK, n_static), lambda r: (r, 0)),
            full((n_static, vocab, L)),
        ],
        out_specs=pl.BlockSpec((SBLK, n_static, L), lambda r: (r, 0, 0)),
        out_shape=jax.ShapeDtypeStruct((B, n_static, L), jnp.float32),
    )(sidx, static_tables)

    return (static_embs,
            jnp.transpose(known_p, (2, 0, 3, 1)),
            jnp.swapaxes(obs_p, 2, 3))
